# baseline (speedup 1.0000x reference)
import functools

import jax
import jax.numpy as jnp
from jax import lax
from jax.experimental import pallas as pl
from jax.experimental.pallas import tpu as pltpu
from jax.experimental.pallas import tpu_sc as plsc

T = 200
B = 4096
E = 2 * B
NW = 32
STEPS_PER_W = 7
TPAD = NW * STEPS_PER_W
LANES = 16
RED_ITERS = E // LANES

_mesh = plsc.VectorSubcoreMesh(core_axis_name="c", subcore_axis_name="s")
_CP = pltpu.CompilerParams(use_tc_tiling_on_sc=False, needs_layout_passes=False)


@functools.partial(
    pl.kernel,
    mesh=_mesh,
    compiler_params=_CP,
    out_type=jax.ShapeDtypeStruct((TPAD * LANES,), jnp.float32),
    scratch_types=[
        pltpu.VMEM((E,), jnp.int32),
        pltpu.VMEM((E,), jnp.float32),
        pltpu.VMEM((STEPS_PER_W * LANES,), jnp.float32),
        pltpu.SemaphoreType.DMA,
    ],
)
def _gather_sum(flat_hbm, steps_hbm, acc_hbm, idx_v, rows_v, buf_v, sem):
    wid = lax.axis_index("s") * 2 + lax.axis_index("c")
    for k in range(STEPS_PER_W):
        t = wid * STEPS_PER_W + k
        pltpu.sync_copy(steps_hbm.at[pl.ds(t * E, E)], idx_v)
        pltpu.async_copy(flat_hbm.at[idx_v], rows_v, sem).wait()

        def red_body(j, acc):
            return acc + rows_v[pl.ds(j * LANES, LANES)]

        acc = lax.fori_loop(0, RED_ITERS, red_body,
                            jnp.zeros((LANES,), jnp.float32))
        buf_v[pl.ds(k * LANES, LANES)] = acc
    pltpu.sync_copy(
        buf_v,
        acc_hbm.at[pl.ds(wid * (STEPS_PER_W * LANES), STEPS_PER_W * LANES)])


@functools.partial(
    pl.kernel,
    mesh=_mesh,
    compiler_params=_CP,
    out_type=jax.ShapeDtypeStruct((TPAD,), jnp.float32),
    scratch_types=[
        pltpu.VMEM((TPAD * LANES,), jnp.float32),
        pltpu.VMEM((LANES,), jnp.float32),
        pltpu.VMEM((TPAD,), jnp.float32),
    ],
)
def _recurrence(acc_hbm, m0_hbm, out_hbm, acc_v, m0_v, out_v):
    wid = lax.axis_index("s") * 2 + lax.axis_index("c")

    @pl.when(wid == 0)
    def _():
        pltpu.sync_copy(acc_hbm, acc_v)
        pltpu.sync_copy(m0_hbm, m0_v)
        lane = lax.iota(jnp.int32, LANES)
        even = (lane & 1) == 0
        lane0 = lane == 0
        m0 = m0_v[...][0]

        def body(t, m):
            row = acc_v[pl.ds(t * LANES, LANES)]
            a = jnp.sum(jnp.where(even, row, 0.0))
            b = jnp.sum(jnp.where(even, 0.0, row))
            m_new = a * m + b
            tt = jnp.full((LANES,), t, jnp.int32)
            plsc.store_scatter(out_v, [tt], jnp.full((LANES,), m_new),
                               mask=lane0)
            return m_new

        lax.fori_loop(0, T, body, m0)
        pltpu.sync_copy(out_v, out_hbm)


def kernel(params, tot_step, M_prev):
    ts = jnp.pad(tot_step.astype(jnp.int32), ((0, TPAD - T), (0, 0)))
    steps2 = (ts[..., None] * 2
              + jnp.arange(2, dtype=jnp.int32)).reshape(-1)
    acc = _gather_sum(params.astype(jnp.float32).reshape(-1), steps2)
    m0 = jnp.broadcast_to(M_prev.astype(jnp.float32), (LANES,))
    out = _recurrence(acc, m0)
    return out[:T].reshape(T, 1)


# speedup vs baseline: 2.8020x; 2.8020x over previous
import functools

import jax
import jax.numpy as jnp
from jax import lax
from jax.experimental import pallas as pl
from jax.experimental.pallas import tpu as pltpu
from jax.experimental.pallas import tpu_sc as plsc

T = 200
B = 4096
N = 1000000
NW = 32
SPW = 7
TPAD = NW * SPW
LANES = 16
UNROLL = 8
RED_ITERS = B // (LANES * UNROLL)

_mesh = plsc.VectorSubcoreMesh(core_axis_name="c", subcore_axis_name="s")
_CP = pltpu.CompilerParams(use_tc_tiling_on_sc=False, needs_layout_passes=False)


def _split_body(p_ref, a_ref, b_ref):
    x = p_ref[...]
    a_ref[...] = x[:, 0]
    b_ref[...] = x[:, 1]


def _split_cols(params):
    rb = 8192
    grid = (N + rb - 1) // rb
    return pl.pallas_call(
        _split_body,
        grid=(grid,),
        in_specs=[pl.BlockSpec((rb, 2), lambda g: (g, 0))],
        out_specs=[pl.BlockSpec((rb,), lambda g: (g,)),
                   pl.BlockSpec((rb,), lambda g: (g,))],
        out_shape=[jax.ShapeDtypeStruct((N,), jnp.float32),
                   jax.ShapeDtypeStruct((N,), jnp.float32)],
    )(params)


def _flat_body(s_ref, o_ref):
    o_ref[...] = s_ref[...].reshape(8 * B)


def _flatten_steps(tot_step):
    return pl.pallas_call(
        _flat_body,
        grid=(T // 8,),
        in_specs=[pl.BlockSpec((8, B), lambda g: (g, 0))],
        out_specs=pl.BlockSpec((8 * B,), lambda g: (g,)),
        out_shape=jax.ShapeDtypeStruct((T * B,), jnp.int32),
    )(tot_step)


@functools.partial(
    pl.kernel,
    mesh=_mesh,
    compiler_params=_CP,
    out_type=jax.ShapeDtypeStruct((TPAD * 2 * LANES,), jnp.float32),
    scratch_types=[
        pltpu.VMEM((B,), jnp.int32),
        pltpu.VMEM((B,), jnp.int32),
        pltpu.VMEM((B,), jnp.float32),
        pltpu.VMEM((B,), jnp.float32),
        pltpu.VMEM((B,), jnp.float32),
        pltpu.VMEM((B,), jnp.float32),
        pltpu.VMEM((SPW * 2 * LANES,), jnp.float32),
        pltpu.SemaphoreType.DMA,
        pltpu.SemaphoreType.DMA,
        pltpu.SemaphoreType.DMA,
        pltpu.SemaphoreType.DMA,
        pltpu.SemaphoreType.DMA,
        pltpu.SemaphoreType.DMA,
    ],
)
def _gather_sum(cola_hbm, colb_hbm, steps_hbm, acc_hbm,
                idx0, idx1, a0, a1, b0, b1, buf_v,
                si0, si1, sa0, sa1, sb0, sb1):
    wid = lax.axis_index("s") * 2 + lax.axis_index("c")
    idx = (idx0, idx1)
    av = (a0, a1)
    bv = (b0, b1)
    sis = (si0, si1)
    sas = (sa0, sa1)
    sbs = (sb0, sb1)

    def tstep(k):
        return wid * SPW + k

    def fire_idx(k, slot):
        @pl.when(tstep(k) < T)
        def _():
            pltpu.async_copy(steps_hbm.at[pl.ds(tstep(k) * B, B)],
                             idx[slot], sis[slot])

    def fire_gathers(k, slot):
        @pl.when(tstep(k) < T)
        def _():
            pltpu.async_copy(cola_hbm.at[idx[slot]], av[slot], sas[slot])
            pltpu.async_copy(colb_hbm.at[idx[slot]], bv[slot], sbs[slot])

    def wait_idx(k, slot):
        @pl.when(tstep(k) < T)
        def _():
            pltpu.make_async_copy(steps_hbm.at[pl.ds(tstep(k) * B, B)],
                                  idx[slot], sis[slot]).wait()

    def reduce(ref):
        zero = jnp.zeros((LANES,), jnp.float32)

        def body(j, accs):
            base = j * (LANES * UNROLL)
            r = list(accs)
            for u in range(UNROLL):
                r[u % 4] = r[u % 4] + ref[pl.ds(base + u * LANES, LANES)]
            return tuple(r)

        accs = lax.fori_loop(0, RED_ITERS, body, (zero, zero, zero, zero))
        return (accs[0] + accs[1]) + (accs[2] + accs[3])

    fire_idx(0, 0)
    fire_idx(1, 1)
    wait_idx(0, 0)
    fire_gathers(0, 0)
    for k in range(SPW):
        cur = k % 2
        nxt = (k + 1) % 2
        if k + 1 < SPW:
            wait_idx(k + 1, nxt)
            fire_gathers(k + 1, nxt)

        @pl.when(tstep(k) < T)
        def _(k=k, cur=cur):
            pltpu.make_async_copy(cola_hbm.at[idx[cur]], av[cur],
                                  sas[cur]).wait()
            buf_v[pl.ds(k * 2 * LANES, LANES)] = reduce(av[cur])
            pltpu.make_async_copy(colb_hbm.at[idx[cur]], bv[cur],
                                  sbs[cur]).wait()
            buf_v[pl.ds(k * 2 * LANES + LANES, LANES)] = reduce(bv[cur])

        if k + 2 < SPW:
            fire_idx(k + 2, cur)
    pltpu.sync_copy(
        buf_v,
        acc_hbm.at[pl.ds(wid * (SPW * 2 * LANES), SPW * 2 * LANES)])


@functools.partial(
    pl.kernel,
    mesh=_mesh,
    compiler_params=_CP,
    out_type=jax.ShapeDtypeStruct((TPAD,), jnp.float32),
    scratch_types=[
        pltpu.VMEM((TPAD * 2 * LANES,), jnp.float32),
        pltpu.VMEM((LANES,), jnp.float32),
        pltpu.VMEM((TPAD,), jnp.float32),
    ],
)
def _recurrence(acc_hbm, m0_hbm, out_hbm, acc_v, m0_v, out_v):
    wid = lax.axis_index("s") * 2 + lax.axis_index("c")

    @pl.when(wid == 0)
    def _():
        pltpu.sync_copy(acc_hbm, acc_v)
        pltpu.sync_copy(m0_hbm, m0_v)
        lane = lax.iota(jnp.int32, LANES)
        lane0 = lane == 0
        m0 = m0_v[...][0]

        def body(t, m):
            a = jnp.sum(acc_v[pl.ds(t * 2 * LANES, LANES)])
            b = jnp.sum(acc_v[pl.ds(t * 2 * LANES + LANES, LANES)])
            m_new = a * m + b
            tt = jnp.full((LANES,), t, jnp.int32)
            plsc.store_scatter(out_v, [tt], jnp.full((LANES,), m_new),
                               mask=lane0)
            return m_new

        lax.fori_loop(0, T, body, m0)
        pltpu.sync_copy(out_v, out_hbm)


def kernel(params, tot_step, M_prev):
    cola, colb = _split_cols(params.astype(jnp.float32))
    steps = _flatten_steps(tot_step.astype(jnp.int32))
    acc = _gather_sum(cola, colb, steps)
    m0 = jnp.broadcast_to(M_prev.astype(jnp.float32), (LANES,))
    out = _recurrence(acc, m0)
    return out[:T].reshape(T, 1)


# speedup vs baseline: 12.9298x; 4.6146x over previous
import functools

import jax
import jax.numpy as jnp
from jax import lax
from jax.experimental import pallas as pl
from jax.experimental.pallas import tpu as pltpu
from jax.experimental.pallas import tpu_sc as plsc

T = 200
B = 4096
N = 1000000
NW = 32
SPW = 7
TPAD = NW * SPW
LANES = 16
UNROLL = 8
RED_ITERS = B // (LANES * UNROLL)

_mesh = plsc.VectorSubcoreMesh(core_axis_name="c", subcore_axis_name="s")
_CP = pltpu.CompilerParams(use_tc_tiling_on_sc=False, needs_layout_passes=False)


def _split_cols(params):
    sel = jnp.eye(2, dtype=jnp.float32)
    return params @ sel[:, 0], params @ sel[:, 1]


def _flat_body(s_ref, o_ref):
    o_ref[...] = s_ref[...].reshape(8 * B)


def _flatten_steps(tot_step):
    return pl.pallas_call(
        _flat_body,
        grid=(T // 8,),
        in_specs=[pl.BlockSpec((8, B), lambda g: (g, 0))],
        out_specs=pl.BlockSpec((8 * B,), lambda g: (g,)),
        out_shape=jax.ShapeDtypeStruct((T * B,), jnp.int32),
    )(tot_step)


@functools.partial(
    pl.kernel,
    mesh=_mesh,
    compiler_params=_CP,
    out_type=jax.ShapeDtypeStruct((TPAD * 2 * LANES,), jnp.float32),
    scratch_types=[
        pltpu.VMEM((B,), jnp.int32),
        pltpu.VMEM((B,), jnp.int32),
        pltpu.VMEM((B,), jnp.float32),
        pltpu.VMEM((B,), jnp.float32),
        pltpu.VMEM((B,), jnp.float32),
        pltpu.VMEM((B,), jnp.float32),
        pltpu.VMEM((SPW * 2 * LANES,), jnp.float32),
        pltpu.SemaphoreType.DMA,
        pltpu.SemaphoreType.DMA,
        pltpu.SemaphoreType.DMA,
        pltpu.SemaphoreType.DMA,
        pltpu.SemaphoreType.DMA,
        pltpu.SemaphoreType.DMA,
    ],
)
def _gather_sum(cola_hbm, colb_hbm, steps_hbm, acc_hbm,
                idx0, idx1, a0, a1, b0, b1, buf_v,
                si0, si1, sa0, sa1, sb0, sb1):
    wid = lax.axis_index("s") * 2 + lax.axis_index("c")
    idx = (idx0, idx1)
    av = (a0, a1)
    bv = (b0, b1)
    sis = (si0, si1)
    sas = (sa0, sa1)
    sbs = (sb0, sb1)

    def tstep(k):
        return wid * SPW + k

    def fire_idx(k, slot):
        @pl.when(tstep(k) < T)
        def _():
            pltpu.async_copy(steps_hbm.at[pl.ds(tstep(k) * B, B)],
                             idx[slot], sis[slot])

    def fire_gathers(k, slot):
        @pl.when(tstep(k) < T)
        def _():
            pltpu.async_copy(cola_hbm.at[idx[slot]], av[slot], sas[slot])
            pltpu.async_copy(colb_hbm.at[idx[slot]], bv[slot], sbs[slot])

    def wait_idx(k, slot):
        @pl.when(tstep(k) < T)
        def _():
            pltpu.make_async_copy(steps_hbm.at[pl.ds(tstep(k) * B, B)],
                                  idx[slot], sis[slot]).wait()

    def reduce(ref):
        zero = jnp.zeros((LANES,), jnp.float32)

        def body(j, accs):
            base = j * (LANES * UNROLL)
            r = list(accs)
            for u in range(UNROLL):
                r[u % 4] = r[u % 4] + ref[pl.ds(base + u * LANES, LANES)]
            return tuple(r)

        accs = lax.fori_loop(0, RED_ITERS, body, (zero, zero, zero, zero))
        return (accs[0] + accs[1]) + (accs[2] + accs[3])

    fire_idx(0, 0)
    fire_idx(1, 1)
    wait_idx(0, 0)
    fire_gathers(0, 0)
    for k in range(SPW):
        cur = k % 2
        nxt = (k + 1) % 2
        if k + 1 < SPW:
            wait_idx(k + 1, nxt)
            fire_gathers(k + 1, nxt)

        @pl.when(tstep(k) < T)
        def _(k=k, cur=cur):
            pltpu.make_async_copy(cola_hbm.at[idx[cur]], av[cur],
                                  sas[cur]).wait()
            buf_v[pl.ds(k * 2 * LANES, LANES)] = reduce(av[cur])
            pltpu.make_async_copy(colb_hbm.at[idx[cur]], bv[cur],
                                  sbs[cur]).wait()
            buf_v[pl.ds(k * 2 * LANES + LANES, LANES)] = reduce(bv[cur])

        if k + 2 < SPW:
            fire_idx(k + 2, cur)
    pltpu.sync_copy(
        buf_v,
        acc_hbm.at[pl.ds(wid * (SPW * 2 * LANES), SPW * 2 * LANES)])


@functools.partial(
    pl.kernel,
    mesh=_mesh,
    compiler_params=_CP,
    out_type=jax.ShapeDtypeStruct((TPAD,), jnp.float32),
    scratch_types=[
        pltpu.VMEM((TPAD * 2 * LANES,), jnp.float32),
        pltpu.VMEM((LANES,), jnp.float32),
        pltpu.VMEM((TPAD,), jnp.float32),
    ],
)
def _recurrence(acc_hbm, m0_hbm, out_hbm, acc_v, m0_v, out_v):
    wid = lax.axis_index("s") * 2 + lax.axis_index("c")

    @pl.when(wid == 0)
    def _():
        pltpu.sync_copy(acc_hbm, acc_v)
        pltpu.sync_copy(m0_hbm, m0_v)
        lane = lax.iota(jnp.int32, LANES)
        lane0 = lane == 0
        m0 = m0_v[...][0]

        def body(t, m):
            a = jnp.sum(acc_v[pl.ds(t * 2 * LANES, LANES)])
            b = jnp.sum(acc_v[pl.ds(t * 2 * LANES + LANES, LANES)])
            m_new = a * m + b
            tt = jnp.full((LANES,), t, jnp.int32)
            plsc.store_scatter(out_v, [tt], jnp.full((LANES,), m_new),
                               mask=lane0)
            return m_new

        lax.fori_loop(0, T, body, m0)
        pltpu.sync_copy(out_v, out_hbm)


def kernel(params, tot_step, M_prev):
    cola, colb = _split_cols(params.astype(jnp.float32))
    steps = _flatten_steps(tot_step.astype(jnp.int32))
    acc = _gather_sum(cola, colb, steps)
    m0 = jnp.broadcast_to(M_prev.astype(jnp.float32), (LANES,))
    out = _recurrence(acc, m0)
    return out[:T].reshape(T, 1)
